# baseline (speedup 1.0000x reference)
import functools
import numpy as np
import jax
import jax.numpy as jnp
from jax import lax
from jax.experimental import pallas as pl
from jax.experimental.pallas import tpu as pltpu
from jax.experimental.pallas import tpu_sc as plsc


def _silu_cst():
    x = np.linspace(-12.0, 12.0, 100001)
    phi = np.exp(-0.5 * x * x) / np.sqrt(2.0 * np.pi)
    s = x / (1.0 + np.exp(-x))
    m2 = np.trapz(s * s * phi, x)
    return float(1.0 / np.sqrt(m2))


_SILU_CST = _silu_cst()

_NC, _NS, _LANES = 2, 16, 16
_NW = _NC * _NS
_NSEG = 5


def _prep_body(nf_ref, ff_ref, wl1_ref, wl2_ref, wmp_ref,
               wftst_ref, wvst_ref, wftrt_ref, wvrt_ref,
               ts_ref, tr_ref, mp_ref):
    f32 = jnp.float32
    nf = nf_ref[...]
    d = nf.shape[1]
    inv_sqrt_d = 1.0 / np.sqrt(float(d))
    ts_ref[:, :d] = jnp.dot(nf, wl1_ref[...], preferred_element_type=f32) * inv_sqrt_d
    tr_ref[:, :d] = jnp.dot(nf, wl2_ref[...], preferred_element_type=f32) * inv_sqrt_d
    ff = ff_ref[...]
    ts_ref[:, d:] = jnp.dot(jnp.dot(ff, wftst_ref[...], preferred_element_type=f32),
                            wvst_ref[...], preferred_element_type=f32)
    tr_ref[:, d:] = jnp.dot(jnp.dot(ff, wftrt_ref[...], preferred_element_type=f32),
                            wvrt_ref[...], preferred_element_type=f32)
    t = jnp.dot(nf, wmp_ref[...], preferred_element_type=f32)
    acc = jnp.zeros((nf.shape[0], 4), f32)
    for j in range(8):
        acc = acc + ff[:, j:j + 1] * t[:, 4 * j:4 * j + 4]
    mp = acc * (0.01 / 32.0)
    lane = lax.broadcasted_iota(jnp.int32, mp.shape, 1)
    mp_ref[...] = jnp.where(lane == 0, 0.0, mp)


def _prep(node_feats, field_feats, w_lin1, w_lin2, w_mp32,
          wfts_t, wvs_t, wftr_t, wvr_t):
    n, d = node_feats.shape
    bn = 2000
    grid = n // bn
    full = lambda s: pl.BlockSpec(s, lambda i: (0, 0))
    rowd = pl.BlockSpec((bn, 2 * d), lambda i: (i, 0))
    return pl.pallas_call(
        _prep_body,
        grid=(grid,),
        in_specs=[
            pl.BlockSpec((bn, d), lambda i: (i, 0)),
            pl.BlockSpec((bn, 8), lambda i: (i, 0)),
            full((d, d)), full((d, d)), full((d, 32)),
            full((8, 8)), full((8, d)), full((8, 8)), full((8, d)),
        ],
        out_specs=[rowd, rowd,
                   pl.BlockSpec((bn, 4), lambda i: (i, 0))],
        out_shape=[
            jax.ShapeDtypeStruct((n, 2 * d), jnp.float32),
            jax.ShapeDtypeStruct((n, 2 * d), jnp.float32),
            jax.ShapeDtypeStruct((n, 4), jnp.float32),
        ],
    )(node_feats, field_feats, w_lin1, w_lin2, w_mp32,
      wfts_t, wvs_t, wftr_t, wvr_t)


def _gather_body(seg_base, epw, cg, nchunks, ts, tr, snd, rcv,
                 gs_out, gr_out,
                 ixs0, ixs1, ixr0, ixr1, bs0, bs1, br0, br1,
                 sgs0, sgs1, sgr0, sgr1, sws0, sws1, swr0, swr1):
    wid = lax.axis_index("s") * _NC + lax.axis_index("c")
    w0 = seg_base + wid * epw

    ixs = (ixs0, ixs1)
    ixr = (ixr0, ixr1)
    bs = (bs0, bs1)
    br = (br0, br1)
    sgs = (sgs0, sgs1)
    sgr = (sgr0, sgr1)
    sws = (sws0, sws1)
    swr = (swr0, swr1)

    def load_idx(i):
        base = w0 + i * cg
        pltpu.sync_copy(snd.at[pl.ds(base, cg)], ixs[i % 2])
        pltpu.sync_copy(rcv.at[pl.ds(base, cg)], ixr[i % 2])

    gd_s = [None, None]
    gd_r = [None, None]
    wd_s = [None, None]
    wd_r = [None, None]

    def start_g(i):
        b = i % 2
        gd_s[b] = pltpu.async_copy(ts.at[ixs[b]], bs[b], sgs[b])
        gd_r[b] = pltpu.async_copy(tr.at[ixr[b]], br[b], sgr[b])

    def start_w(i):
        b = i % 2
        wbase = wid * epw + i * cg
        wd_s[b] = pltpu.async_copy(bs[b], gs_out.at[pl.ds(wbase, cg)], sws[b])
        wd_r[b] = pltpu.async_copy(br[b], gr_out.at[pl.ds(wbase, cg)], swr[b])

    load_idx(0)
    start_g(0)
    for i in range(nchunks):
        b = i % 2
        if i + 1 < nchunks:
            load_idx(i + 1)
            if i >= 1:
                wd_s[(i + 1) % 2].wait()
                wd_r[(i + 1) % 2].wait()
            start_g(i + 1)
        gd_s[b].wait()
        gd_r[b].wait()
        start_w(i)
    wd_s[(nchunks - 1) % 2].wait()
    wd_r[(nchunks - 1) % 2].wait()
    if nchunks >= 2:
        wd_s[nchunks % 2].wait()
        wd_r[nchunks % 2].wait()


def _gather(ts, tr, sender, receiver, seg_base, eseg):
    n, d2 = ts.shape
    epw = eseg // _NW
    cg = 80
    nchunks = epw // cg
    mesh = plsc.VectorSubcoreMesh(core_axis_name="c", subcore_axis_name="s",
                                  num_cores=_NC, num_subcores=_NS)
    f32 = jnp.float32
    kern = pl.kernel(
        functools.partial(_gather_body, seg_base, epw, cg, nchunks),
        out_type=[jax.ShapeDtypeStruct((eseg, d2), f32) for _ in range(2)],
        mesh=mesh,
        scratch_types=(
            [pltpu.VMEM((cg,), jnp.int32) for _ in range(4)]
            + [pltpu.VMEM((cg, d2), f32) for _ in range(4)]
            + [pltpu.SemaphoreType.DMA for _ in range(8)]
        ),
    )
    return kern(ts, tr, sender, receiver)


def _edge_body(eft_ref, gs_ref, gr_ref,
               w0_ref, w1_ref, w2_ref, w3_ref,
               pe_ref):
    f32 = jnp.float32
    bf = jnp.bfloat16
    d = w3_ref.shape[1]
    eft = eft_ref[...].astype(bf)
    h = jax.nn.silu(lax.dot_general(eft, w0_ref[...], (((0,), (0,)), ((), ())),
                                    preferred_element_type=f32)
                    * (1.0 / np.sqrt(8.0))) * _SILU_CST
    h = jax.nn.silu(jnp.dot(h.astype(bf), w1_ref[...], preferred_element_type=f32)
                    * 0.125) * _SILU_CST
    h = jax.nn.silu(jnp.dot(h.astype(bf), w2_ref[...], preferred_element_type=f32)
                    * 0.125) * _SILU_CST
    tp = jnp.dot(h.astype(bf), w3_ref[...], preferred_element_type=f32) * 0.125
    c = gs_ref[:, :d] + gr_ref[:, :d]
    u = gs_ref[:, d:] + gr_ref[:, d:]
    k = 0.1 / (8.0 * np.sqrt(float(d)))
    prod = tp * c * u
    ones8 = jnp.ones((d, 8), f32)
    p8 = jnp.dot(prod, ones8, preferred_element_type=f32) * k
    pe_ref[...] = p8[:, 0]


def _edge(eft, gs, gr, w0, w1, w2, w3, seg_off):
    eseg, d2 = gs.shape
    d = d2 // 2
    h = w1.shape[0]
    be = 512
    grid = eseg // be
    ob = seg_off // be
    full = lambda s: pl.BlockSpec(s, lambda i: (0, 0))
    return pl.pallas_call(
        _edge_body,
        grid=(grid,),
        in_specs=[
            pl.BlockSpec((8, be), lambda i: (0, i + ob)),
            pl.BlockSpec((be, d2), lambda i: (i, 0)),
            pl.BlockSpec((be, d2), lambda i: (i, 0)),
            full((8, h)), full((h, h)), full((h, h)), full((h, d)),
        ],
        out_specs=pl.BlockSpec((be,), lambda i: (i,)),
        out_shape=jax.ShapeDtypeStruct((eseg,), jnp.float32),
    )(eft, gs, gr, w0, w1, w2, w3)


def _finish_body(pe_ref, ea_ref, out_ref):
    out_ref[...] = pe_ref[...] * ea_ref[...]


def _finish(pe2, ea2):
    r, c = pe2.shape
    return pl.pallas_call(
        _finish_body,
        in_specs=[pl.BlockSpec((r, c), lambda: (0, 0)),
                  pl.BlockSpec((r, c), lambda: (0, 0))],
        out_specs=pl.BlockSpec((r, c), lambda: (0, 0)),
        out_shape=jax.ShapeDtypeStruct((r, c), jnp.float32),
    )(pe2, ea2)


def _scatter_body(n, epw, cs, pe, snd, rcv, out32, pv, iv_s, iv_r, acc_t):
    cid = lax.axis_index("c")
    sid = lax.axis_index("s")
    wid = sid * _NC + cid
    zero16 = jnp.zeros((_LANES,), jnp.float32)

    @pl.loop(0, n // _LANES)
    def _zero(j):
        acc_t[pl.ds(j * _LANES, _LANES)] = zero16

    @pl.loop(0, epw // cs)
    def _chunk(i):
        base = wid * epw + i * cs
        pltpu.sync_copy(pe.at[pl.ds(base, cs)], pv)
        pltpu.sync_copy(snd.at[pl.ds(base, cs)], iv_s)
        pltpu.sync_copy(rcv.at[pl.ds(base, cs)], iv_r)

        @pl.loop(0, cs // _LANES)
        def _vec(j):
            sl = pl.ds(j * _LANES, _LANES)
            pvec = pv[sl]
            plsc.addupdate_scatter(acc_t, [iv_r[sl]], pvec)
            plsc.addupdate_scatter(acc_t, [iv_s[sl]], -pvec)

    pltpu.sync_copy(acc_t, out32.at[wid])


def _scatter(pe, sender, receiver, n):
    e = sender.shape[0]
    epw = e // _NW
    cs = 2000
    mesh = plsc.VectorSubcoreMesh(core_axis_name="c", subcore_axis_name="s",
                                  num_cores=_NC, num_subcores=_NS)
    f32 = jnp.float32
    kern = pl.kernel(
        functools.partial(_scatter_body, n, epw, cs),
        out_type=jax.ShapeDtypeStruct((_NW, n), f32),
        mesh=mesh,
        compiler_params=pltpu.CompilerParams(needs_layout_passes=False),
        scratch_types=[
            pltpu.VMEM((cs,), f32),
            pltpu.VMEM((cs,), jnp.int32),
            pltpu.VMEM((cs,), jnp.int32),
            pltpu.VMEM((n,), f32),
        ],
    )
    return kern(pe, sender, receiver)


def _combine_body(ch_ref, out_ref):
    out_ref[...] = jnp.sum(ch_ref[...], axis=0)


def _combine(ch32, n):
    return pl.pallas_call(
        _combine_body,
        in_specs=[pl.BlockSpec((_NW, n), lambda: (0, 0))],
        out_specs=pl.BlockSpec((n,), lambda: (0,)),
        out_shape=jax.ShapeDtypeStruct((n,), jnp.float32),
    )(ch32)


def kernel(node_attrs, node_feats, edge_attrs, edge_feats, edge_index,
           field_feats, W_lin1, W_lin2, W_mlp0, W_mlp1, W_mlp2, W_mlp3,
           W_vs, W_vr, W_fts, W_ftr, W_mp):
    n, d = node_feats.shape
    e = edge_index.shape[1]
    eseg = e // _NSEG
    sender = edge_index[0]
    receiver = edge_index[1]
    w_mp32 = W_mp.reshape(d, 32)
    eft = edge_feats.T
    bf = jnp.bfloat16
    w0b = W_mlp0.astype(bf)
    w1b = W_mlp1.astype(bf)
    w2b = W_mlp2.astype(bf)
    w3b = W_mlp3.astype(bf)

    ts, tr, mp0 = _prep(node_feats, field_feats, W_lin1, W_lin2,
                        w_mp32, W_fts.T, W_vs.T, W_ftr.T, W_vr.T)
    pes = []
    for s in range(_NSEG):
        gs, gr = _gather(ts, tr, sender, receiver, s * eseg, eseg)
        pe_s = _edge(eft, gs, gr, w0b, w1b, w2b, w3b, s * eseg)
        pes.append(pe_s)
    pe_raw = jnp.concatenate(pes, axis=0)
    pf2 = _finish(pe_raw.reshape(e // 128, 128),
                  edge_attrs.reshape(e // 128, 128))
    pe = pf2.reshape(e)
    ch32 = _scatter(pe, sender, receiver, n)
    charges = _combine(ch32, n)
    mp = mp0.at[:, 0].set(charges)
    return (mp[:, None, :], pe[:, None])


# speedup vs baseline: 1.2211x; 1.2211x over previous
import functools
import numpy as np
import jax
import jax.numpy as jnp
from jax import lax
from jax.experimental import pallas as pl
from jax.experimental.pallas import tpu as pltpu
from jax.experimental.pallas import tpu_sc as plsc


def _silu_cst():
    x = np.linspace(-12.0, 12.0, 100001)
    phi = np.exp(-0.5 * x * x) / np.sqrt(2.0 * np.pi)
    s = x / (1.0 + np.exp(-x))
    m2 = np.trapz(s * s * phi, x)
    return float(1.0 / np.sqrt(m2))


_SILU_CST = _silu_cst()

_NC, _NS, _LANES = 2, 16, 16
_NW = _NC * _NS
_NSEG = 5


def _prep_body(nf_ref, ff_ref, wl1_ref, wl2_ref, wmp_ref,
               wftst_ref, wvst_ref, wftrt_ref, wvrt_ref,
               ts_ref, tr_ref, mp_ref):
    f32 = jnp.float32
    nf = nf_ref[...]
    d = nf.shape[1]
    inv_sqrt_d = 1.0 / np.sqrt(float(d))
    bf = jnp.bfloat16
    u16 = jnp.uint16
    u32 = jnp.uint32

    def pack(lo, hi):
        lo16 = lax.bitcast_convert_type(lo.astype(bf), u16).astype(u32)
        hi16 = lax.bitcast_convert_type(hi.astype(bf), u16).astype(u32)
        return lax.bitcast_convert_type(lo16 | (hi16 << 16), f32)

    lin1 = jnp.dot(nf, wl1_ref[...], preferred_element_type=f32) * inv_sqrt_d
    lin2 = jnp.dot(nf, wl2_ref[...], preferred_element_type=f32) * inv_sqrt_d
    ff = ff_ref[...]
    us = jnp.dot(jnp.dot(ff, wftst_ref[...], preferred_element_type=f32),
                 wvst_ref[...], preferred_element_type=f32)
    ur = jnp.dot(jnp.dot(ff, wftrt_ref[...], preferred_element_type=f32),
                 wvrt_ref[...], preferred_element_type=f32)
    ts_ref[...] = pack(lin1, us)
    tr_ref[...] = pack(lin2, ur)
    t = jnp.dot(nf, wmp_ref[...], preferred_element_type=f32)
    acc = jnp.zeros((nf.shape[0], 4), f32)
    for j in range(8):
        acc = acc + ff[:, j:j + 1] * t[:, 4 * j:4 * j + 4]
    mp = acc * (0.01 / 32.0)
    lane = lax.broadcasted_iota(jnp.int32, mp.shape, 1)
    mp_ref[...] = jnp.where(lane == 0, 0.0, mp)


def _prep(node_feats, field_feats, w_lin1, w_lin2, w_mp32,
          wfts_t, wvs_t, wftr_t, wvr_t):
    n, d = node_feats.shape
    bn = 2000
    grid = n // bn
    full = lambda s: pl.BlockSpec(s, lambda i: (0, 0))
    rowd = pl.BlockSpec((bn, d), lambda i: (i, 0))
    return pl.pallas_call(
        _prep_body,
        grid=(grid,),
        in_specs=[
            pl.BlockSpec((bn, d), lambda i: (i, 0)),
            pl.BlockSpec((bn, 8), lambda i: (i, 0)),
            full((d, d)), full((d, d)), full((d, 32)),
            full((8, 8)), full((8, d)), full((8, 8)), full((8, d)),
        ],
        out_specs=[rowd, rowd,
                   pl.BlockSpec((bn, 4), lambda i: (i, 0))],
        out_shape=[
            jax.ShapeDtypeStruct((n, d), jnp.float32),
            jax.ShapeDtypeStruct((n, d), jnp.float32),
            jax.ShapeDtypeStruct((n, 4), jnp.float32),
        ],
    )(node_feats, field_feats, w_lin1, w_lin2, w_mp32,
      wfts_t, wvs_t, wftr_t, wvr_t)


def _gather_body(seg_base, epw, cg, nchunks, ts, tr, snd, rcv,
                 gs_out, gr_out,
                 ixs0, ixs1, ixr0, ixr1, bs0, bs1, br0, br1,
                 sgs0, sgs1, sgr0, sgr1, sws0, sws1, swr0, swr1):
    wid = lax.axis_index("s") * _NC + lax.axis_index("c")
    w0 = seg_base + wid * epw

    ixs = (ixs0, ixs1)
    ixr = (ixr0, ixr1)
    bs = (bs0, bs1)
    br = (br0, br1)
    sgs = (sgs0, sgs1)
    sgr = (sgr0, sgr1)
    sws = (sws0, sws1)
    swr = (swr0, swr1)

    def load_idx(i):
        base = w0 + i * cg
        pltpu.sync_copy(snd.at[pl.ds(base, cg)], ixs[i % 2])
        pltpu.sync_copy(rcv.at[pl.ds(base, cg)], ixr[i % 2])

    gd_s = [None, None]
    gd_r = [None, None]
    wd_s = [None, None]
    wd_r = [None, None]

    def start_g(i):
        b = i % 2
        gd_s[b] = pltpu.async_copy(ts.at[ixs[b]], bs[b], sgs[b])
        gd_r[b] = pltpu.async_copy(tr.at[ixr[b]], br[b], sgr[b])

    def start_w(i):
        b = i % 2
        wbase = wid * epw + i * cg
        wd_s[b] = pltpu.async_copy(bs[b], gs_out.at[pl.ds(wbase, cg)], sws[b])
        wd_r[b] = pltpu.async_copy(br[b], gr_out.at[pl.ds(wbase, cg)], swr[b])

    load_idx(0)
    start_g(0)
    for i in range(nchunks):
        b = i % 2
        if i + 1 < nchunks:
            load_idx(i + 1)
            if i >= 1:
                wd_s[(i + 1) % 2].wait()
                wd_r[(i + 1) % 2].wait()
            start_g(i + 1)
        gd_s[b].wait()
        gd_r[b].wait()
        start_w(i)
    wd_s[(nchunks - 1) % 2].wait()
    wd_r[(nchunks - 1) % 2].wait()
    if nchunks >= 2:
        wd_s[nchunks % 2].wait()
        wd_r[nchunks % 2].wait()


def _gather(ts, tr, sender, receiver, seg_base, eseg):
    n, d = ts.shape
    epw = eseg // _NW
    cg = 200
    nchunks = epw // cg
    mesh = plsc.VectorSubcoreMesh(core_axis_name="c", subcore_axis_name="s",
                                  num_cores=_NC, num_subcores=_NS)
    f32 = jnp.float32
    kern = pl.kernel(
        functools.partial(_gather_body, seg_base, epw, cg, nchunks),
        out_type=[jax.ShapeDtypeStruct((eseg, d), f32) for _ in range(2)],
        mesh=mesh,
        scratch_types=(
            [pltpu.VMEM((cg,), jnp.int32) for _ in range(4)]
            + [pltpu.VMEM((cg, d), f32) for _ in range(4)]
            + [pltpu.SemaphoreType.DMA for _ in range(8)]
        ),
    )
    return kern(ts, tr, sender, receiver)


def _edge_body(eft_ref, gs_ref, gr_ref,
               w0_ref, w1_ref, w2_ref, w3_ref,
               pe_ref):
    f32 = jnp.float32
    bf = jnp.bfloat16
    d = w3_ref.shape[1]
    eft = eft_ref[...].astype(bf)
    h = jax.nn.silu(lax.dot_general(eft, w0_ref[...], (((0,), (0,)), ((), ())),
                                    preferred_element_type=f32)
                    * (1.0 / np.sqrt(8.0))) * _SILU_CST
    h = jax.nn.silu(jnp.dot(h.astype(bf), w1_ref[...], preferred_element_type=f32)
                    * 0.125) * _SILU_CST
    h = jax.nn.silu(jnp.dot(h.astype(bf), w2_ref[...], preferred_element_type=f32)
                    * 0.125) * _SILU_CST
    tp = jnp.dot(h.astype(bf), w3_ref[...], preferred_element_type=f32) * 0.125
    u32 = jnp.uint32
    gsw = lax.bitcast_convert_type(gs_ref[...], u32)
    grw = lax.bitcast_convert_type(gr_ref[...], u32)
    himask = jnp.uint32(0xFFFF0000)
    c = (lax.bitcast_convert_type(gsw << 16, f32)
         + lax.bitcast_convert_type(grw << 16, f32))
    u = (lax.bitcast_convert_type(gsw & himask, f32)
         + lax.bitcast_convert_type(grw & himask, f32))
    k = 0.1 / (8.0 * np.sqrt(float(d)))
    prod = tp * c * u
    ones8 = jnp.ones((d, 8), f32)
    p8 = jnp.dot(prod, ones8, preferred_element_type=f32) * k
    pe_ref[...] = p8[:, 0]


def _edge(eft, gs, gr, w0, w1, w2, w3, seg_off):
    eseg, d = gs.shape
    h = w1.shape[0]
    be = 512
    grid = eseg // be
    ob = seg_off // be
    full = lambda s: pl.BlockSpec(s, lambda i: (0, 0))
    return pl.pallas_call(
        _edge_body,
        grid=(grid,),
        in_specs=[
            pl.BlockSpec((8, be), lambda i: (0, i + ob)),
            pl.BlockSpec((be, d), lambda i: (i, 0)),
            pl.BlockSpec((be, d), lambda i: (i, 0)),
            full((8, h)), full((h, h)), full((h, h)), full((h, d)),
        ],
        out_specs=pl.BlockSpec((be,), lambda i: (i,)),
        out_shape=jax.ShapeDtypeStruct((eseg,), jnp.float32),
    )(eft, gs, gr, w0, w1, w2, w3)


def _finish_body(pe_ref, ea_ref, out_ref):
    out_ref[...] = pe_ref[...] * ea_ref[...]


def _finish(pe2, ea2):
    r, c = pe2.shape
    return pl.pallas_call(
        _finish_body,
        in_specs=[pl.BlockSpec((r, c), lambda: (0, 0)),
                  pl.BlockSpec((r, c), lambda: (0, 0))],
        out_specs=pl.BlockSpec((r, c), lambda: (0, 0)),
        out_shape=jax.ShapeDtypeStruct((r, c), jnp.float32),
    )(pe2, ea2)


def _scatter_body(n, epw, cs, pe, snd, rcv, out32, pv, iv_s, iv_r, acc_t):
    cid = lax.axis_index("c")
    sid = lax.axis_index("s")
    wid = sid * _NC + cid
    zero16 = jnp.zeros((_LANES,), jnp.float32)

    @pl.loop(0, n // _LANES)
    def _zero(j):
        acc_t[pl.ds(j * _LANES, _LANES)] = zero16

    @pl.loop(0, epw // cs)
    def _chunk(i):
        base = wid * epw + i * cs
        pltpu.sync_copy(pe.at[pl.ds(base, cs)], pv)
        pltpu.sync_copy(snd.at[pl.ds(base, cs)], iv_s)
        pltpu.sync_copy(rcv.at[pl.ds(base, cs)], iv_r)

        @pl.loop(0, cs // _LANES)
        def _vec(j):
            sl = pl.ds(j * _LANES, _LANES)
            pvec = pv[sl]
            plsc.addupdate_scatter(acc_t, [iv_r[sl]], pvec)
            plsc.addupdate_scatter(acc_t, [iv_s[sl]], -pvec)

    pltpu.sync_copy(acc_t, out32.at[wid])


def _scatter(pe, sender, receiver, n):
    e = sender.shape[0]
    epw = e // _NW
    cs = 2000
    mesh = plsc.VectorSubcoreMesh(core_axis_name="c", subcore_axis_name="s",
                                  num_cores=_NC, num_subcores=_NS)
    f32 = jnp.float32
    kern = pl.kernel(
        functools.partial(_scatter_body, n, epw, cs),
        out_type=jax.ShapeDtypeStruct((_NW, n), f32),
        mesh=mesh,
        compiler_params=pltpu.CompilerParams(needs_layout_passes=False),
        scratch_types=[
            pltpu.VMEM((cs,), f32),
            pltpu.VMEM((cs,), jnp.int32),
            pltpu.VMEM((cs,), jnp.int32),
            pltpu.VMEM((n,), f32),
        ],
    )
    return kern(pe, sender, receiver)


def _combine_body(ch_ref, out_ref):
    out_ref[...] = jnp.sum(ch_ref[...], axis=0)


def _combine(ch32, n):
    return pl.pallas_call(
        _combine_body,
        in_specs=[pl.BlockSpec((_NW, n), lambda: (0, 0))],
        out_specs=pl.BlockSpec((n,), lambda: (0,)),
        out_shape=jax.ShapeDtypeStruct((n,), jnp.float32),
    )(ch32)


def kernel(node_attrs, node_feats, edge_attrs, edge_feats, edge_index,
           field_feats, W_lin1, W_lin2, W_mlp0, W_mlp1, W_mlp2, W_mlp3,
           W_vs, W_vr, W_fts, W_ftr, W_mp):
    n, d = node_feats.shape
    e = edge_index.shape[1]
    eseg = e // _NSEG
    sender = edge_index[0]
    receiver = edge_index[1]
    w_mp32 = W_mp.reshape(d, 32)
    eft = edge_feats.T
    bf = jnp.bfloat16
    w0b = W_mlp0.astype(bf)
    w1b = W_mlp1.astype(bf)
    w2b = W_mlp2.astype(bf)
    w3b = W_mlp3.astype(bf)

    ts, tr, mp0 = _prep(node_feats, field_feats, W_lin1, W_lin2,
                        w_mp32, W_fts.T, W_vs.T, W_ftr.T, W_vr.T)
    pes = []
    for s in range(_NSEG):
        gs, gr = _gather(ts, tr, sender, receiver, s * eseg, eseg)
        pe_s = _edge(eft, gs, gr, w0b, w1b, w2b, w3b, s * eseg)
        pes.append(pe_s)
    pe_raw = jnp.concatenate(pes, axis=0)
    pf2 = _finish(pe_raw.reshape(e // 128, 128),
                  edge_attrs.reshape(e // 128, 128))
    pe = pf2.reshape(e)
    ch32 = _scatter(pe, sender, receiver, n)
    charges = _combine(ch32, n)
    mp = mp0.at[:, 0].set(charges)
    return (mp[:, None, :], pe[:, None])


# speedup vs baseline: 2.2565x; 1.8479x over previous
import functools
import numpy as np
import jax
import jax.numpy as jnp
from jax import lax
from jax.experimental import pallas as pl
from jax.experimental.pallas import tpu as pltpu
from jax.experimental.pallas import tpu_sc as plsc


def _silu_cst():
    x = np.linspace(-12.0, 12.0, 100001)
    phi = np.exp(-0.5 * x * x) / np.sqrt(2.0 * np.pi)
    s = x / (1.0 + np.exp(-x))
    m2 = np.trapz(s * s * phi, x)
    return float(1.0 / np.sqrt(m2))


_SILU_CST = _silu_cst()

_NC, _NS, _LANES = 2, 16, 16
_NW = _NC * _NS
_NSEG = 5


def _prep_body(nf_ref, ff_ref, wl1_ref, wl2_ref, wmp_ref,
               wftst_ref, wvst_ref, wftrt_ref, wvrt_ref,
               ts_ref, tr_ref, mp_ref):
    f32 = jnp.float32
    bf = jnp.bfloat16
    nf = nf_ref[...]
    d = nf.shape[1]
    inv_sqrt_d = 1.0 / np.sqrt(float(d))
    u16 = jnp.uint16
    u32 = jnp.uint32

    def pack(lo, hi):
        lo16 = lax.bitcast_convert_type(lo.astype(bf), u16).astype(u32)
        hi16 = lax.bitcast_convert_type(hi.astype(bf), u16).astype(u32)
        return lax.bitcast_convert_type(lo16 | (hi16 << 16), f32)

    nfb = nf.astype(bf)
    lin1 = jnp.dot(nfb, wl1_ref[...].astype(bf),
                   preferred_element_type=f32) * inv_sqrt_d
    lin2 = jnp.dot(nfb, wl2_ref[...].astype(bf),
                   preferred_element_type=f32) * inv_sqrt_d
    ff = ff_ref[...]
    us = jnp.dot(jnp.dot(ff, wftst_ref[...], preferred_element_type=f32),
                 wvst_ref[...], preferred_element_type=f32)
    ur = jnp.dot(jnp.dot(ff, wftrt_ref[...], preferred_element_type=f32),
                 wvrt_ref[...], preferred_element_type=f32)
    ts_ref[...] = pack(lin1, us)
    tr_ref[...] = pack(lin2, ur)
    t = jnp.dot(nf, wmp_ref[...], preferred_element_type=f32)
    acc = jnp.zeros((nf.shape[0], 4), f32)
    for j in range(8):
        acc = acc + ff[:, j:j + 1] * t[:, 4 * j:4 * j + 4]
    mp = acc * (0.01 / 32.0)
    lane = lax.broadcasted_iota(jnp.int32, mp.shape, 1)
    mp_ref[...] = jnp.where(lane == 0, 0.0, mp)


def _prep(node_feats, field_feats, w_lin1, w_lin2, w_mp32,
          wfts_t, wvs_t, wftr_t, wvr_t):
    n, d = node_feats.shape
    bn = 2000
    grid = n // bn
    full = lambda s: pl.BlockSpec(s, lambda i: (0, 0))
    rowd = pl.BlockSpec((bn, d), lambda i: (i, 0))
    return pl.pallas_call(
        _prep_body,
        grid=(grid,),
        in_specs=[
            pl.BlockSpec((bn, d), lambda i: (i, 0)),
            pl.BlockSpec((bn, 8), lambda i: (i, 0)),
            full((d, d)), full((d, d)), full((d, 32)),
            full((8, 8)), full((8, d)), full((8, 8)), full((8, d)),
        ],
        out_specs=[rowd, rowd,
                   pl.BlockSpec((bn, 4), lambda i: (i, 0))],
        out_shape=[
            jax.ShapeDtypeStruct((n, d), jnp.float32),
            jax.ShapeDtypeStruct((n, d), jnp.float32),
            jax.ShapeDtypeStruct((n, 4), jnp.float32),
        ],
    )(node_feats, field_feats, w_lin1, w_lin2, w_mp32,
      wfts_t, wvs_t, wftr_t, wvr_t)


def _gather_body(seg_base, epw, cg, nchunks, ts, tr, snd, rcv,
                 gs_out, gr_out,
                 ixs0, ixs1, ixr0, ixr1, bs0, bs1, br0, br1,
                 sgs0, sgs1, sgr0, sgr1, sws0, sws1, swr0, swr1):
    wid = lax.axis_index("s") * _NC + lax.axis_index("c")
    w0 = seg_base + wid * epw

    ixs = (ixs0, ixs1)
    ixr = (ixr0, ixr1)
    bs = (bs0, bs1)
    br = (br0, br1)
    sgs = (sgs0, sgs1)
    sgr = (sgr0, sgr1)
    sws = (sws0, sws1)
    swr = (swr0, swr1)

    def load_idx(i):
        base = w0 + i * cg
        pltpu.sync_copy(snd.at[pl.ds(base, cg)], ixs[i % 2])
        pltpu.sync_copy(rcv.at[pl.ds(base, cg)], ixr[i % 2])

    gd_s = [None, None]
    gd_r = [None, None]
    wd_s = [None, None]
    wd_r = [None, None]

    def start_g(i):
        b = i % 2
        gd_s[b] = pltpu.async_copy(ts.at[ixs[b]], bs[b], sgs[b])
        gd_r[b] = pltpu.async_copy(tr.at[ixr[b]], br[b], sgr[b])

    def start_w(i):
        b = i % 2
        wbase = wid * epw + i * cg
        wd_s[b] = pltpu.async_copy(bs[b], gs_out.at[pl.ds(wbase, cg)], sws[b])
        wd_r[b] = pltpu.async_copy(br[b], gr_out.at[pl.ds(wbase, cg)], swr[b])

    load_idx(0)
    start_g(0)
    for i in range(nchunks):
        b = i % 2
        if i + 1 < nchunks:
            load_idx(i + 1)
            if i >= 1:
                wd_s[(i + 1) % 2].wait()
                wd_r[(i + 1) % 2].wait()
            start_g(i + 1)
        gd_s[b].wait()
        gd_r[b].wait()
        start_w(i)
    wd_s[(nchunks - 1) % 2].wait()
    wd_r[(nchunks - 1) % 2].wait()
    if nchunks >= 2:
        wd_s[nchunks % 2].wait()
        wd_r[nchunks % 2].wait()


def _gather(ts, tr, sender, receiver, seg_base, eseg):
    n, d = ts.shape
    epw = eseg // _NW
    cg = 200
    nchunks = epw // cg
    mesh = plsc.VectorSubcoreMesh(core_axis_name="c", subcore_axis_name="s",
                                  num_cores=_NC, num_subcores=_NS)
    f32 = jnp.float32
    kern = pl.kernel(
        functools.partial(_gather_body, seg_base, epw, cg, nchunks),
        out_type=[jax.ShapeDtypeStruct((eseg, d), f32) for _ in range(2)],
        mesh=mesh,
        scratch_types=(
            [pltpu.VMEM((cg,), jnp.int32) for _ in range(4)]
            + [pltpu.VMEM((cg, d), f32) for _ in range(4)]
            + [pltpu.SemaphoreType.DMA for _ in range(8)]
        ),
    )
    return kern(ts, tr, sender, receiver)


def _edge_body(eft_ref, gs_ref, gr_ref,
               w0_ref, w1_ref, w2_ref, w3_ref,
               pe_ref):
    f32 = jnp.float32
    bf = jnp.bfloat16
    d = w3_ref.shape[1]
    eft = eft_ref[...].astype(bf)
    h = jax.nn.silu(lax.dot_general(eft, w0_ref[...], (((0,), (0,)), ((), ())),
                                    preferred_element_type=f32)
                    * (1.0 / np.sqrt(8.0))) * _SILU_CST
    h = jax.nn.silu(jnp.dot(h.astype(bf), w1_ref[...], preferred_element_type=f32)
                    * 0.125) * _SILU_CST
    h = jax.nn.silu(jnp.dot(h.astype(bf), w2_ref[...], preferred_element_type=f32)
                    * 0.125) * _SILU_CST
    tp = jnp.dot(h.astype(bf), w3_ref[...], preferred_element_type=f32) * 0.125
    u32 = jnp.uint32
    gsw = lax.bitcast_convert_type(gs_ref[...], u32)
    grw = lax.bitcast_convert_type(gr_ref[...], u32)
    himask = jnp.uint32(0xFFFF0000)
    c = (lax.bitcast_convert_type(gsw << 16, f32)
         + lax.bitcast_convert_type(grw << 16, f32))
    u = (lax.bitcast_convert_type(gsw & himask, f32)
         + lax.bitcast_convert_type(grw & himask, f32))
    prodb = (tp * c * u).astype(bf)
    onest = jnp.ones((d, 8), bf)
    p8t = lax.dot_general(onest, prodb, (((0,), (1,)), ((), ())),
                          preferred_element_type=f32)
    pe_ref[0] = p8t[0:1, :]


def _edge(eft, gs, gr, w0, w1, w2, w3, seg_off):
    eseg, d = gs.shape
    h = w1.shape[0]
    be = 12800
    grid = eseg // be
    ob = seg_off // be
    full = lambda s: pl.BlockSpec(s, lambda i: (0, 0))
    out = pl.pallas_call(
        _edge_body,
        grid=(grid,),
        in_specs=[
            pl.BlockSpec((8, be), lambda i: (0, i + ob)),
            pl.BlockSpec((be, d), lambda i: (i, 0)),
            pl.BlockSpec((be, d), lambda i: (i, 0)),
            full((8, h)), full((h, h)), full((h, h)), full((h, d)),
        ],
        out_specs=pl.BlockSpec((1, 1, be), lambda i: (i, 0, 0)),
        out_shape=jax.ShapeDtypeStruct((grid, 1, be), jnp.float32),
    )(eft, gs, gr, w0, w1, w2, w3)
    return out.reshape(eseg)


def _finish_body(k, pe_ref, ea_ref, out_ref):
    out_ref[...] = pe_ref[...] * ea_ref[...] * k


def _finish(pe2, ea2, k):
    r, c = pe2.shape
    return pl.pallas_call(
        functools.partial(_finish_body, k),
        in_specs=[pl.BlockSpec((r, c), lambda: (0, 0)),
                  pl.BlockSpec((r, c), lambda: (0, 0))],
        out_specs=pl.BlockSpec((r, c), lambda: (0, 0)),
        out_shape=jax.ShapeDtypeStruct((r, c), jnp.float32),
    )(pe2, ea2)


def _scatter_body(n, epw, cs, pe, snd, rcv, out32, pv, iv_s, iv_r, acc_t):
    cid = lax.axis_index("c")
    sid = lax.axis_index("s")
    wid = sid * _NC + cid
    zero16 = jnp.zeros((_LANES,), jnp.float32)

    @pl.loop(0, n // _LANES)
    def _zero(j):
        acc_t[pl.ds(j * _LANES, _LANES)] = zero16

    @pl.loop(0, epw // cs)
    def _chunk(i):
        base = wid * epw + i * cs
        pltpu.sync_copy(pe.at[pl.ds(base, cs)], pv)
        pltpu.sync_copy(snd.at[pl.ds(base, cs)], iv_s)
        pltpu.sync_copy(rcv.at[pl.ds(base, cs)], iv_r)

        @pl.loop(0, cs // _LANES)
        def _vec(j):
            sl = pl.ds(j * _LANES, _LANES)
            pvec = pv[sl]
            plsc.addupdate_scatter(acc_t, [iv_r[sl]], pvec)
            plsc.addupdate_scatter(acc_t, [iv_s[sl]], -pvec)

    pltpu.sync_copy(acc_t, out32.at[wid])


def _scatter(pe, sender, receiver, n):
    e = sender.shape[0]
    epw = e // _NW
    cs = 2000
    mesh = plsc.VectorSubcoreMesh(core_axis_name="c", subcore_axis_name="s",
                                  num_cores=_NC, num_subcores=_NS)
    f32 = jnp.float32
    kern = pl.kernel(
        functools.partial(_scatter_body, n, epw, cs),
        out_type=jax.ShapeDtypeStruct((_NW, n), f32),
        mesh=mesh,
        compiler_params=pltpu.CompilerParams(needs_layout_passes=False),
        scratch_types=[
            pltpu.VMEM((cs,), f32),
            pltpu.VMEM((cs,), jnp.int32),
            pltpu.VMEM((cs,), jnp.int32),
            pltpu.VMEM((n,), f32),
        ],
    )
    return kern(pe, sender, receiver)


def _combine_body(ch_ref, out_ref):
    out_ref[...] = jnp.sum(ch_ref[...], axis=0)


def _combine(ch32, n):
    return pl.pallas_call(
        _combine_body,
        in_specs=[pl.BlockSpec((_NW, n), lambda: (0, 0))],
        out_specs=pl.BlockSpec((n,), lambda: (0,)),
        out_shape=jax.ShapeDtypeStruct((n,), jnp.float32),
    )(ch32)


def kernel(node_attrs, node_feats, edge_attrs, edge_feats, edge_index,
           field_feats, W_lin1, W_lin2, W_mlp0, W_mlp1, W_mlp2, W_mlp3,
           W_vs, W_vr, W_fts, W_ftr, W_mp):
    n, d = node_feats.shape
    e = edge_index.shape[1]
    eseg = e // _NSEG
    sender = edge_index[0]
    receiver = edge_index[1]
    w_mp32 = W_mp.reshape(d, 32)
    eft = edge_feats.T
    bf = jnp.bfloat16
    w0b = W_mlp0.astype(bf)
    w1b = W_mlp1.astype(bf)
    w2b = W_mlp2.astype(bf)
    w3b = W_mlp3.astype(bf)

    ts, tr, mp0 = _prep(node_feats, field_feats, W_lin1, W_lin2,
                        w_mp32, W_fts.T, W_vs.T, W_ftr.T, W_vr.T)
    pes = []
    for s in range(_NSEG):
        gs, gr = _gather(ts, tr, sender, receiver, s * eseg, eseg)
        pe_s = _edge(eft, gs, gr, w0b, w1b, w2b, w3b, s * eseg)
        pes.append(pe_s)
    pe_raw = jnp.concatenate(pes, axis=0)
    k = 0.1 / (8.0 * np.sqrt(float(d)))
    pf2 = _finish(pe_raw.reshape(e // 128, 128),
                  edge_attrs[:, 0].reshape(e // 128, 128), k)
    pe = pf2.reshape(e)
    ch32 = _scatter(pe, sender, receiver, n)
    charges = _combine(ch32, n)
    mp = mp0.at[:, 0].set(charges)
    return (mp[:, None, :], pe[:, None])
